# initial kernel scaffold (unmeasured)
import jax
import jax.numpy as jnp
from jax import lax
from jax.experimental import pallas as pl
from jax.experimental.pallas import tpu as pltpu

N_DEV = 4


def kernel(x, W1, W2):
    m, _ = x.shape
    d = W1.shape[1]
    n = W2.shape[1]
    chunk = m // N_DEV

    def body(x_ref, w1_ref, w2_ref, out_ref, h_ref, rs_recv,
             rs_send_sems, rs_recv_sems, ag_send_sems, ag_recv_sems):
        my = lax.axis_index("i")
        left = (my + N_DEV - 1) % N_DEV
        right = (my + 1) % N_DEV

        barrier_sem = pltpu.get_barrier_semaphore()
        for nbr in [left, right]:
            pl.semaphore_signal(
                barrier_sem, inc=1,
                device_id=(nbr,), device_id_type=pl.DeviceIdType.MESH,
            )
        pl.semaphore_wait(barrier_sem, 2)

        h_ref[...] = jnp.dot(
            x_ref[...], w1_ref[...], preferred_element_type=jnp.float32
        )

        for s in range(N_DEV - 1):
            c_send = (my + 2 * N_DEV - 1 - s) % N_DEV
            rdma = pltpu.make_async_remote_copy(
                src_ref=h_ref.at[pl.ds(c_send * chunk, chunk)],
                dst_ref=rs_recv.at[s],
                send_sem=rs_send_sems.at[s],
                recv_sem=rs_recv_sems.at[s],
                device_id=(right,),
                device_id_type=pl.DeviceIdType.MESH,
            )
            rdma.start()
            rdma.wait()
            c_recv = (my + 2 * N_DEV - 2 - s) % N_DEV
            h_ref[pl.ds(c_recv * chunk, chunk), :] += rs_recv[s]

        out_ref[pl.ds(my * chunk, chunk), :] = jnp.dot(
            h_ref[pl.ds(my * chunk, chunk), :], w2_ref[...],
            preferred_element_type=jnp.float32,
        )

        for h in range(N_DEV - 1):
            c = (my + N_DEV - h) % N_DEV
            rdma = pltpu.make_async_remote_copy(
                src_ref=out_ref.at[pl.ds(c * chunk, chunk)],
                dst_ref=out_ref.at[pl.ds(c * chunk, chunk)],
                send_sem=ag_send_sems.at[h],
                recv_sem=ag_recv_sems.at[h],
                device_id=(right,),
                device_id_type=pl.DeviceIdType.MESH,
            )
            rdma.start()
            rdma.wait()

    return pl.pallas_call(
        body,
        out_shape=jax.ShapeDtypeStruct((m, n), jnp.float32),
        in_specs=[
            pl.BlockSpec(memory_space=pltpu.VMEM),
            pl.BlockSpec(memory_space=pltpu.VMEM),
            pl.BlockSpec(memory_space=pltpu.VMEM),
        ],
        out_specs=pl.BlockSpec(memory_space=pltpu.VMEM),
        scratch_shapes=[
            pltpu.VMEM((m, d), jnp.float32),
            pltpu.VMEM((N_DEV - 1, chunk, d), jnp.float32),
            pltpu.SemaphoreType.DMA((N_DEV - 1,)),
            pltpu.SemaphoreType.DMA((N_DEV - 1,)),
            pltpu.SemaphoreType.DMA((N_DEV - 1,)),
            pltpu.SemaphoreType.DMA((N_DEV - 1,)),
        ],
        compiler_params=pltpu.CompilerParams(collective_id=0),
    )(x, W1, W2)


# baseline (device time: 90620 ns/iter reference)
import jax
import jax.numpy as jnp
from jax import lax
from jax.experimental import pallas as pl
from jax.experimental.pallas import tpu as pltpu

N_DEV = 4


def kernel(x, W1, W2):
    m, _ = x.shape
    d = W1.shape[1]
    n = W2.shape[1]
    chunk = m // N_DEV

    def body(x_ref, w1_ref, w2_ref, out_ref, h_ref, rs_recv,
             rs_send_sems, rs_recv_sems, ag_send_sems, ag_recv_sems):
        my = lax.axis_index("i")
        left = (my + N_DEV - 1) % N_DEV
        right = (my + 1) % N_DEV

        barrier_sem = pltpu.get_barrier_semaphore()
        for nbr in [left, right]:
            pl.semaphore_signal(
                barrier_sem, inc=1,
                device_id=(nbr,), device_id_type=pl.DeviceIdType.MESH,
            )
        pl.semaphore_wait(barrier_sem, 2)

        h_ref[...] = jnp.dot(
            x_ref[...], w1_ref[...], preferred_element_type=jnp.float32
        )

        for s in range(N_DEV - 1):
            c_send = (my + 2 * N_DEV - 1 - s) % N_DEV
            rdma = pltpu.make_async_remote_copy(
                src_ref=h_ref.at[pl.ds(c_send * chunk, chunk)],
                dst_ref=rs_recv.at[s],
                send_sem=rs_send_sems.at[s],
                recv_sem=rs_recv_sems.at[s],
                device_id=(right,),
                device_id_type=pl.DeviceIdType.MESH,
            )
            rdma.start()
            rdma.wait()
            c_recv = (my + 2 * N_DEV - 2 - s) % N_DEV
            h_ref[pl.ds(c_recv * chunk, chunk), :] += rs_recv[s]

        out_ref[pl.ds(my * chunk, chunk), :] = jnp.dot(
            h_ref[pl.ds(my * chunk, chunk), :], w2_ref[...],
            preferred_element_type=jnp.float32,
        )

        for h in range(N_DEV - 1):
            c = (my + N_DEV - h) % N_DEV
            rdma = pltpu.make_async_remote_copy(
                src_ref=h_ref.at[pl.ds(c * chunk, chunk)],
                dst_ref=h_ref.at[pl.ds(c * chunk, chunk)],
                send_sem=ag_send_sems.at[h],
                recv_sem=ag_recv_sems.at[h],
                device_id=(right,),
                device_id_type=pl.DeviceIdType.MESH,
            )
            rdma.start()
            rdma.wait()
            c_recv = (my + 2 * N_DEV - 1 - h) % N_DEV
            out_ref[pl.ds(c_recv * chunk, chunk), :] = jnp.dot(
                h_ref[pl.ds(c_recv * chunk, chunk), :], w2_ref[...],
                preferred_element_type=jnp.float32,
            )

    return pl.pallas_call(
        body,
        out_shape=jax.ShapeDtypeStruct((m, n), jnp.float32),
        in_specs=[
            pl.BlockSpec(memory_space=pltpu.VMEM),
            pl.BlockSpec(memory_space=pltpu.VMEM),
            pl.BlockSpec(memory_space=pltpu.VMEM),
        ],
        out_specs=pl.BlockSpec(memory_space=pltpu.VMEM),
        scratch_shapes=[
            pltpu.VMEM((m, d), jnp.float32),
            pltpu.VMEM((N_DEV - 1, chunk, d), jnp.float32),
            pltpu.SemaphoreType.DMA((N_DEV - 1,)),
            pltpu.SemaphoreType.DMA((N_DEV - 1,)),
            pltpu.SemaphoreType.DMA((N_DEV - 1,)),
            pltpu.SemaphoreType.DMA((N_DEV - 1,)),
        ],
        compiler_params=pltpu.CompilerParams(collective_id=0),
    )(x, W1, W2)


# device time: 54778 ns/iter; 1.6543x vs baseline; 1.6543x over previous
import jax
import jax.numpy as jnp
from jax import lax
from jax.experimental import pallas as pl
from jax.experimental.pallas import tpu as pltpu

N_DEV = 4


def kernel(x, W1, W2):
    m, _ = x.shape
    d = W1.shape[1]
    n = W2.shape[1]
    chunk = m // N_DEV
    half = chunk // 2

    def body(x_ref, w1_ref, w2_ref, out_ref, h_ref,
             recv_a, recv_b,
             rsa_send, rsa_recv, rsb_send, rsb_recv,
             aga_send, aga_recv, agb_send, agb_recv):
        my = lax.axis_index("i")
        left = (my + N_DEV - 1) % N_DEV
        right = (my + 1) % N_DEV

        def row_a(c):
            return ((c % N_DEV) * chunk)

        def row_b(c):
            return ((c % N_DEV) * chunk) + half

        def gemm1_tile(row):
            h_ref[pl.ds(row, half), :] = jnp.dot(
                x_ref[pl.ds(row, half), :], w1_ref[...],
                preferred_element_type=jnp.float32,
            )

        def gemm2_tile(row):
            out_ref[pl.ds(row, half), :] = jnp.dot(
                h_ref[pl.ds(row, half), :], w2_ref[...],
                preferred_element_type=jnp.float32,
            )

        def send(src_row, dst_row, dst_dev, sems_s, sems_r, slot, to_recvbuf):
            dst = (recv_a if to_recvbuf == "a" else
                   recv_b if to_recvbuf == "b" else None)
            rdma = pltpu.make_async_remote_copy(
                src_ref=h_ref.at[pl.ds(src_row, half)],
                dst_ref=(dst.at[slot] if dst is not None
                         else h_ref.at[pl.ds(dst_row, half)]),
                send_sem=sems_s.at[slot],
                recv_sem=sems_r.at[slot],
                device_id=(dst_dev,),
                device_id_type=pl.DeviceIdType.MESH,
            )
            rdma.start()
            return rdma

        barrier_sem = pltpu.get_barrier_semaphore()
        for nbr in [left, right]:
            pl.semaphore_signal(
                barrier_sem, inc=1,
                device_id=(nbr,), device_id_type=pl.DeviceIdType.MESH,
            )
        pl.semaphore_wait(barrier_sem, 2)

        gemm1_tile(row_a(my + 3))
        gemm1_tile(row_b(my + 1))
        ra = send(row_a(my + 3), 0, right, rsa_send, rsa_recv, 0, "a")
        rb = send(row_b(my + 1), 0, left, rsb_send, rsb_recv, 0, "b")
        gemm1_tile(row_a(my + 2))
        gemm1_tile(row_b(my + 2))
        gemm1_tile(row_a(my + 1))
        gemm1_tile(row_b(my + 3))
        gemm1_tile(row_a(my))
        gemm1_tile(row_b(my))

        for s in range(N_DEV - 1):
            ra.wait()
            rb.wait()
            c_ra = my + 2 * N_DEV - 2 - s
            c_rb = my + 2 + s
            h_ref[pl.ds(row_a(c_ra), half), :] += recv_a[s]
            h_ref[pl.ds(row_b(c_rb), half), :] += recv_b[s]
            if s < N_DEV - 2:
                ra = send(row_a(c_ra), 0, right,
                          rsa_send, rsa_recv, s + 1, "a")
                rb = send(row_b(c_rb), 0, left,
                          rsb_send, rsb_recv, s + 1, "b")

        ga = send(row_a(my), row_a(my), right, aga_send, aga_recv, 0, None)
        gb = send(row_b(my), row_b(my), left, agb_send, agb_recv, 0, None)
        gemm2_tile(row_a(my))
        gemm2_tile(row_b(my))

        for hh in range(N_DEV - 1):
            ga.wait()
            gb.wait()
            c_ga = my + 2 * N_DEV - 1 - hh
            c_gb = my + 1 + hh
            if hh < N_DEV - 2:
                ga = send(row_a(c_ga), row_a(c_ga), right,
                          aga_send, aga_recv, hh + 1, None)
                gb = send(row_b(c_gb), row_b(c_gb), left,
                          agb_send, agb_recv, hh + 1, None)
            gemm2_tile(row_a(c_ga))
            gemm2_tile(row_b(c_gb))

    return pl.pallas_call(
        body,
        out_shape=jax.ShapeDtypeStruct((m, n), jnp.float32),
        in_specs=[
            pl.BlockSpec(memory_space=pltpu.VMEM),
            pl.BlockSpec(memory_space=pltpu.VMEM),
            pl.BlockSpec(memory_space=pltpu.VMEM),
        ],
        out_specs=pl.BlockSpec(memory_space=pltpu.VMEM),
        scratch_shapes=[
            pltpu.VMEM((m, d), jnp.float32),
            pltpu.VMEM((N_DEV - 1, half, d), jnp.float32),
            pltpu.VMEM((N_DEV - 1, half, d), jnp.float32),
            pltpu.SemaphoreType.DMA((N_DEV - 1,)),
            pltpu.SemaphoreType.DMA((N_DEV - 1,)),
            pltpu.SemaphoreType.DMA((N_DEV - 1,)),
            pltpu.SemaphoreType.DMA((N_DEV - 1,)),
            pltpu.SemaphoreType.DMA((N_DEV - 1,)),
            pltpu.SemaphoreType.DMA((N_DEV - 1,)),
            pltpu.SemaphoreType.DMA((N_DEV - 1,)),
            pltpu.SemaphoreType.DMA((N_DEV - 1,)),
        ],
        compiler_params=pltpu.CompilerParams(collective_id=0),
    )(x, W1, W2)


# device time: 48004 ns/iter; 1.8878x vs baseline; 1.1411x over previous
import jax
import jax.numpy as jnp
from jax import lax
from jax.experimental import pallas as pl
from jax.experimental.pallas import tpu as pltpu

N_DEV = 4
SUB = 2


def kernel(x, W1, W2):
    m, _ = x.shape
    d = W1.shape[1]
    n = W2.shape[1]
    chunk = m // N_DEV
    half = chunk // 2
    sub = half // SUB
    nslots = (N_DEV - 1) * SUB

    def body(x_ref, w1_ref, w2_ref, out_ref, h_ref,
             recv_a, recv_b,
             rsa_send, rsa_recv, rsb_send, rsb_recv,
             aga_send, aga_recv, agb_send, agb_recv):
        my = lax.axis_index("i")
        left = (my + N_DEV - 1) % N_DEV
        right = (my + 1) % N_DEV

        def row_a(c, k=0):
            return ((c % N_DEV) * chunk) + k * sub

        def row_b(c, k=0):
            return ((c % N_DEV) * chunk) + half + k * sub

        def gemm1_tile(row):
            h_ref[pl.ds(row, half), :] = jnp.dot(
                x_ref[pl.ds(row, half), :], w1_ref[...],
                preferred_element_type=jnp.float32,
            )

        def gemm2_piece(row):
            out_ref[pl.ds(row, sub), :] = jnp.dot(
                h_ref[pl.ds(row, sub), :], w2_ref[...],
                preferred_element_type=jnp.float32,
            )

        def send(src_row, dst_dev, sems_s, sems_r, slot, recvbuf, dst_row):
            rdma = pltpu.make_async_remote_copy(
                src_ref=h_ref.at[pl.ds(src_row, sub)],
                dst_ref=(recvbuf.at[slot] if recvbuf is not None
                         else h_ref.at[pl.ds(dst_row, sub)]),
                send_sem=sems_s.at[slot],
                recv_sem=sems_r.at[slot],
                device_id=(dst_dev,),
                device_id_type=pl.DeviceIdType.MESH,
            )
            rdma.start()
            return rdma

        barrier_sem = pltpu.get_barrier_semaphore()
        for nbr in [left, right]:
            pl.semaphore_signal(
                barrier_sem, inc=1,
                device_id=(nbr,), device_id_type=pl.DeviceIdType.MESH,
            )
        pl.semaphore_wait(barrier_sem, 2)

        gemm1_tile(row_a(my + 3))
        gemm1_tile(row_b(my + 1))
        ra = [None] * nslots
        rb = [None] * nslots
        for k in range(SUB):
            ra[k] = send(row_a(my + 3, k), right,
                         rsa_send, rsa_recv, k, recv_a, 0)
            rb[k] = send(row_b(my + 1, k), left,
                         rsb_send, rsb_recv, k, recv_b, 0)
        gemm1_tile(row_a(my + 2))
        gemm1_tile(row_b(my + 2))
        gemm1_tile(row_a(my + 1))
        gemm1_tile(row_b(my + 3))
        gemm1_tile(row_a(my))
        gemm1_tile(row_b(my))

        for s in range(N_DEV - 1):
            c_ra = my + 2 * N_DEV - 2 - s
            c_rb = my + 2 + s
            for k in range(SUB):
                slot = s * SUB + k
                ra[slot].wait()
                h_ref[pl.ds(row_a(c_ra, k), sub), :] += recv_a[slot]
                rb[slot].wait()
                h_ref[pl.ds(row_b(c_rb, k), sub), :] += recv_b[slot]
                if s < N_DEV - 2:
                    nslot = slot + SUB
                    ra[nslot] = send(row_a(c_ra, k), right,
                                     rsa_send, rsa_recv, nslot, recv_a, 0)
                    rb[nslot] = send(row_b(c_rb, k), left,
                                     rsb_send, rsb_recv, nslot, recv_b, 0)

        ga = [None] * nslots
        gb = [None] * nslots
        for k in range(SUB):
            ga[k] = send(row_a(my, k), right,
                         aga_send, aga_recv, k, None, row_a(my, k))
            gb[k] = send(row_b(my, k), left,
                         agb_send, agb_recv, k, None, row_b(my, k))
        gemm2_piece(row_a(my, 0))
        gemm2_piece(row_a(my, 1))
        gemm2_piece(row_b(my, 0))
        gemm2_piece(row_b(my, 1))

        for hh in range(N_DEV - 1):
            c_ga = my + 2 * N_DEV - 1 - hh
            c_gb = my + 1 + hh
            for k in range(SUB):
                slot = hh * SUB + k
                ga[slot].wait()
                if hh < N_DEV - 2:
                    nslot = slot + SUB
                    ga[nslot] = send(row_a(c_ga, k), right, aga_send,
                                     aga_recv, nslot, None, row_a(c_ga, k))
                gemm2_piece(row_a(c_ga, k))
                gb[slot].wait()
                if hh < N_DEV - 2:
                    nslot = slot + SUB
                    gb[nslot] = send(row_b(c_gb, k), left, agb_send,
                                     agb_recv, nslot, None, row_b(c_gb, k))
                gemm2_piece(row_b(c_gb, k))

    return pl.pallas_call(
        body,
        out_shape=jax.ShapeDtypeStruct((m, n), jnp.float32),
        in_specs=[
            pl.BlockSpec(memory_space=pltpu.VMEM),
            pl.BlockSpec(memory_space=pltpu.VMEM),
            pl.BlockSpec(memory_space=pltpu.VMEM),
        ],
        out_specs=pl.BlockSpec(memory_space=pltpu.VMEM),
        scratch_shapes=[
            pltpu.VMEM((m, d), jnp.float32),
            pltpu.VMEM((6, 128, d), jnp.float32),
            pltpu.VMEM((6, 128, d), jnp.float32),
            pltpu.SemaphoreType.DMA((6,)),
            pltpu.SemaphoreType.DMA((6,)),
            pltpu.SemaphoreType.DMA((6,)),
            pltpu.SemaphoreType.DMA((6,)),
            pltpu.SemaphoreType.DMA((6,)),
            pltpu.SemaphoreType.DMA((6,)),
            pltpu.SemaphoreType.DMA((6,)),
            pltpu.SemaphoreType.DMA((6,)),
        ],
        compiler_params=pltpu.CompilerParams(collective_id=0),
    )(x, W1, W2)


# device time: 46414 ns/iter; 1.9524x vs baseline; 1.0343x over previous
import jax
import jax.numpy as jnp
from jax import lax
from jax.experimental import pallas as pl
from jax.experimental.pallas import tpu as pltpu

N_DEV = 4
SUB = 2


def kernel(x, W1, W2):
    m, _ = x.shape
    d = W1.shape[1]
    n = W2.shape[1]
    chunk = m // N_DEV
    half = chunk // 2
    sub = half // SUB
    nslots = (N_DEV - 1) * SUB

    def body(x_ref, w1_ref, w2_ref, out_ref, h_ref,
             recv_a, recv_b,
             rsa_send, rsa_recv, rsb_send, rsb_recv,
             aga_send, aga_recv, agb_send, agb_recv):
        my = lax.axis_index("i")
        left = (my + N_DEV - 1) % N_DEV
        right = (my + 1) % N_DEV

        def row_a(c, k=0):
            return ((c % N_DEV) * chunk) + k * sub

        def row_b(c, k=0):
            return ((c % N_DEV) * chunk) + half + k * sub

        def gemm1_tile(row, rows=half):
            h_ref[pl.ds(row, rows), :] = jnp.dot(
                x_ref[pl.ds(row, rows), :], w1_ref[...],
                preferred_element_type=jnp.float32,
            )

        def gemm2_piece(row):
            out_ref[pl.ds(row, sub), :] = jnp.dot(
                h_ref[pl.ds(row, sub), :], w2_ref[...],
                preferred_element_type=jnp.float32,
            )

        def send(src_row, dst_dev, sems_s, sems_r, slot, recvbuf, dst_row):
            rdma = pltpu.make_async_remote_copy(
                src_ref=h_ref.at[pl.ds(src_row, sub)],
                dst_ref=(recvbuf.at[slot] if recvbuf is not None
                         else h_ref.at[pl.ds(dst_row, sub)]),
                send_sem=sems_s.at[slot],
                recv_sem=sems_r.at[slot],
                device_id=(dst_dev,),
                device_id_type=pl.DeviceIdType.MESH,
            )
            rdma.start()
            return rdma

        barrier_sem = pltpu.get_barrier_semaphore()
        for nbr in [left, right]:
            pl.semaphore_signal(
                barrier_sem, inc=1,
                device_id=(nbr,), device_id_type=pl.DeviceIdType.MESH,
            )
        pl.semaphore_wait(barrier_sem, 2)

        ra = [None] * nslots
        rb = [None] * nslots
        for k in range(SUB):
            gemm1_tile(row_a(my + 3, k), sub)
            ra[k] = send(row_a(my + 3, k), right,
                         rsa_send, rsa_recv, k, recv_a, 0)
            gemm1_tile(row_b(my + 1, k), sub)
            rb[k] = send(row_b(my + 1, k), left,
                         rsb_send, rsb_recv, k, recv_b, 0)
        gemm1_tile(row_a(my + 2))
        gemm1_tile(row_b(my + 2))
        gemm1_tile(row_a(my + 1))
        gemm1_tile(row_b(my + 3))
        gemm1_tile(row_a(my))
        gemm1_tile(row_b(my))

        ga = [None] * nslots
        gb = [None] * nslots
        for s in range(N_DEV - 1):
            c_ra = my + 2 * N_DEV - 2 - s
            c_rb = my + 2 + s
            for k in range(SUB):
                slot = s * SUB + k
                ra[slot].wait()
                h_ref[pl.ds(row_a(c_ra, k), sub), :] += recv_a[slot]
                if s < N_DEV - 2:
                    ra[slot + SUB] = send(row_a(c_ra, k), right, rsa_send,
                                          rsa_recv, slot + SUB, recv_a, 0)
                else:
                    ga[k] = send(row_a(my, k), right,
                                 aga_send, aga_recv, k, None, row_a(my, k))
                rb[slot].wait()
                h_ref[pl.ds(row_b(c_rb, k), sub), :] += recv_b[slot]
                if s < N_DEV - 2:
                    rb[slot + SUB] = send(row_b(c_rb, k), left, rsb_send,
                                          rsb_recv, slot + SUB, recv_b, 0)
                else:
                    gb[k] = send(row_b(my, k), left,
                                 agb_send, agb_recv, k, None, row_b(my, k))

        gemm2_piece(row_a(my, 0))
        gemm2_piece(row_a(my, 1))
        gemm2_piece(row_b(my, 0))
        gemm2_piece(row_b(my, 1))

        for hh in range(N_DEV - 1):
            c_ga = my + 2 * N_DEV - 1 - hh
            c_gb = my + 1 + hh
            for k in range(SUB):
                slot = hh * SUB + k
                ga[slot].wait()
                if hh < N_DEV - 2:
                    nslot = slot + SUB
                    ga[nslot] = send(row_a(c_ga, k), right, aga_send,
                                     aga_recv, nslot, None, row_a(c_ga, k))
                gemm2_piece(row_a(c_ga, k))
                gb[slot].wait()
                if hh < N_DEV - 2:
                    nslot = slot + SUB
                    gb[nslot] = send(row_b(c_gb, k), left, agb_send,
                                     agb_recv, nslot, None, row_b(c_gb, k))
                gemm2_piece(row_b(c_gb, k))

    return pl.pallas_call(
        body,
        out_shape=jax.ShapeDtypeStruct((m, n), jnp.float32),
        in_specs=[
            pl.BlockSpec(memory_space=pltpu.VMEM),
            pl.BlockSpec(memory_space=pltpu.VMEM),
            pl.BlockSpec(memory_space=pltpu.VMEM),
        ],
        out_specs=pl.BlockSpec(memory_space=pltpu.VMEM),
        scratch_shapes=[
            pltpu.VMEM((m, d), jnp.float32),
            pltpu.VMEM((6, 128, d), jnp.float32),
            pltpu.VMEM((6, 128, d), jnp.float32),
            pltpu.SemaphoreType.DMA((6,)),
            pltpu.SemaphoreType.DMA((6,)),
            pltpu.SemaphoreType.DMA((6,)),
            pltpu.SemaphoreType.DMA((6,)),
            pltpu.SemaphoreType.DMA((6,)),
            pltpu.SemaphoreType.DMA((6,)),
            pltpu.SemaphoreType.DMA((6,)),
            pltpu.SemaphoreType.DMA((6,)),
        ],
        compiler_params=pltpu.CompilerParams(collective_id=0),
    )(x, W1, W2)
